# baseline (speedup 1.0000x reference)
import jax
import jax.numpy as jnp
from jax.experimental import pallas as pl
from jax.experimental.pallas import tpu as pltpu

B, L, V = 8, 512, 16
D, K, P, H = 32, 8192, 16, 256
PH = L // P
PF = 512 // P
BV = B * V
N = BV * PF
RB = 256
NBLK = N // RB
TOPK = 5

_pallas_call = pl.pallas_call


def _revin(x2_ref, xn_ref, mu_ref, sig_ref):
    x2 = x2_ref[...]
    mu = jnp.mean(x2, axis=1, keepdims=True)
    var = jnp.mean((x2 - mu) ** 2, axis=1, keepdims=True)
    sig = jnp.sqrt(var + 1e-5)
    xn_ref[...] = (x2 - mu) / sig
    mu_ref[...] = mu
    sig_ref[...] = sig


def _dotb(a, w, dims=(((1,), (0,)), ((), ()))):
    return jax.lax.dot_general(a.astype(jnp.bfloat16), w.astype(jnp.bfloat16),
                               dims, preferred_element_type=jnp.float32)


def _front(patches_ref, Wenc_ref, benc_ref, lng_ref, lnb_ref, W1_ref, b1_ref,
           Wm_ref, bm_ref, W2_ref, b2_ref, cb_ref, zp_ref, c2_ref):
    z_his = _dotb(patches_ref[...], Wenc_ref[...]) + benc_ref[...]
    m2 = jnp.mean(z_his, axis=1, keepdims=True)
    v2 = jnp.mean((z_his - m2) ** 2, axis=1, keepdims=True)
    h = (z_his - m2) / jnp.sqrt(v2 + 1e-5) * lng_ref[...] + lnb_ref[...]
    h = h.reshape(BV, PH, D).transpose(0, 2, 1).reshape(BV * D, PH)
    h = jax.nn.relu(_dotb(h, W1_ref[...]) + b1_ref[...])
    h = jax.nn.relu(_dotb(h, Wm_ref[...]) + bm_ref[...])
    h = _dotb(h, W2_ref[...]) + b2_ref[...]
    zp_ref[...] = h.reshape(BV, D, PF).transpose(0, 2, 1).reshape(N, D)
    cb = cb_ref[...]
    c2_ref[...] = jnp.sum(cb * cb, axis=1, keepdims=True)


def _main(zp_ref, cb_ref, c2_ref, mu_ref, sig_ref, Wf_ref, bf_ref, fg_ref,
          fb_ref, Wd_ref, bd_ref, logits_ref, recon_ref):
    z = zp_ref[...]
    cb = cb_ref[...]
    zc = _dotb(z, cb, (((1,), (1,)), ((), ())))
    z2 = jnp.sum(z * z, axis=1, keepdims=True)
    dist = z2 + c2_ref[...] - 2.0 * zc
    logits_ref[...] = -dist
    v1 = jnp.min(dist, axis=1, keepdims=True)
    v = v1
    for _ in range(TOPK - 1):
        v = jnp.min(jnp.where(dist > v, dist, jnp.inf), axis=1, keepdims=True)
    wraw = jnp.where(dist <= v, jnp.exp(v1 - dist), 0.0)
    denom = jnp.sum(wraw, axis=1, keepdims=True)
    w = wraw / denom
    z_code = jax.lax.dot_general(w, cb, (((1,), (0,)), ((), ())),
                                 precision=jax.lax.Precision.HIGHEST)
    f = z_code + jax.nn.relu(_dotb(z, Wf_ref[...]) + bf_ref[...])
    m = jnp.mean(f, axis=1, keepdims=True)
    va = jnp.mean((f - m) ** 2, axis=1, keepdims=True)
    fz = (f - m) / jnp.sqrt(va + 1e-5) * fg_ref[...] + fb_ref[...]
    rp = _dotb(fz, Wd_ref[...]) + bd_ref[...]
    recon_ref[...] = rp * sig_ref[...] + mu_ref[...]


def kernel(x, codebook, W_enc, b_enc, ln_g, ln_b, W1, b1, Wmid, bmid, W2, b2,
           Wf, bf, fuse_g, fuse_b, W_dec, b_dec):
    r1 = lambda a: a.reshape(1, -1)
    x2 = jnp.transpose(x, (0, 2, 1)).reshape(BV, L)
    xn, mu, sig = _pallas_call(
        _revin,
        out_shape=[
            jax.ShapeDtypeStruct((BV, L), jnp.float32),
            jax.ShapeDtypeStruct((BV, 1), jnp.float32),
            jax.ShapeDtypeStruct((BV, 1), jnp.float32),
        ],
    )(x2)
    patches = xn.reshape(BV * PH, P)
    zp, c2col = _pallas_call(
        _front,
        out_shape=[
            jax.ShapeDtypeStruct((N, D), jnp.float32),
            jax.ShapeDtypeStruct((K, 1), jnp.float32),
        ],
    )(patches, W_enc, r1(b_enc), r1(ln_g), r1(ln_b), W1, r1(b1), Wmid,
      r1(bmid), W2, r1(b2), codebook)
    c2row = c2col.reshape(1, K)
    mu_rep = jnp.repeat(mu, PF, axis=0)
    sig_rep = jnp.repeat(sig, PF, axis=0)
    logits, recon_pre = _pallas_call(
        _main,
        grid=(NBLK,),
        in_specs=[
            pl.BlockSpec((RB, D), lambda i: (i, 0)),
            pl.BlockSpec((K, D), lambda i: (0, 0)),
            pl.BlockSpec((1, K), lambda i: (0, 0)),
            pl.BlockSpec((RB, 1), lambda i: (i, 0)),
            pl.BlockSpec((RB, 1), lambda i: (i, 0)),
            pl.BlockSpec((D, D), lambda i: (0, 0)),
            pl.BlockSpec((1, D), lambda i: (0, 0)),
            pl.BlockSpec((1, D), lambda i: (0, 0)),
            pl.BlockSpec((1, D), lambda i: (0, 0)),
            pl.BlockSpec((D, P), lambda i: (0, 0)),
            pl.BlockSpec((1, P), lambda i: (0, 0)),
        ],
        out_specs=[
            pl.BlockSpec((RB, K), lambda i: (i, 0)),
            pl.BlockSpec((RB, P), lambda i: (i, 0)),
        ],
        out_shape=[
            jax.ShapeDtypeStruct((N, K), jnp.float32),
            jax.ShapeDtypeStruct((N, P), jnp.float32),
        ],
        compiler_params=pltpu.CompilerParams(
            dimension_semantics=("parallel",)),
    )(zp, codebook, c2row, mu_rep, sig_rep, Wf, r1(bf), r1(fuse_g),
      r1(fuse_b), W_dec, r1(b_dec))
    recon = recon_pre.reshape(B, V, L).transpose(0, 2, 1)
    return recon, logits.reshape(B, V, PF, K)


# speedup vs baseline: 1.9242x; 1.9242x over previous
import jax
import jax.numpy as jnp
from jax.experimental import pallas as pl
from jax.experimental.pallas import tpu as pltpu

B, L, V = 8, 512, 16
D, K, P, H = 32, 8192, 16, 256
PH = L // P
PF = 512 // P
BV = B * V
N = BV * PF
RB = 256
NBLK = N // RB
TOPK = 5

_pallas_call = pl.pallas_call


def _revin(x2_ref, xn_ref, mu_ref, sig_ref):
    x2 = x2_ref[...]
    mu = jnp.mean(x2, axis=1, keepdims=True)
    var = jnp.mean((x2 - mu) ** 2, axis=1, keepdims=True)
    sig = jnp.sqrt(var + 1e-5)
    xn_ref[...] = (x2 - mu) / sig
    mu_ref[...] = mu
    sig_ref[...] = sig


def _dotb(a, w, dims=(((1,), (0,)), ((), ()))):
    return jax.lax.dot_general(a.astype(jnp.bfloat16), w.astype(jnp.bfloat16),
                               dims, preferred_element_type=jnp.float32)


def _front(patches_ref, Wenc_ref, benc_ref, lng_ref, lnb_ref, W1_ref, b1_ref,
           Wm_ref, bm_ref, W2_ref, b2_ref, cb_ref, zp_ref, c2_ref):
    z_his = _dotb(patches_ref[...], Wenc_ref[...]) + benc_ref[...]
    m2 = jnp.mean(z_his, axis=1, keepdims=True)
    v2 = jnp.mean((z_his - m2) ** 2, axis=1, keepdims=True)
    h = (z_his - m2) / jnp.sqrt(v2 + 1e-5) * lng_ref[...] + lnb_ref[...]
    h = h.reshape(BV, PH, D).transpose(0, 2, 1).reshape(BV * D, PH)
    h = jax.nn.relu(_dotb(h, W1_ref[...]) + b1_ref[...])
    h = jax.nn.relu(_dotb(h, Wm_ref[...]) + bm_ref[...])
    h = _dotb(h, W2_ref[...]) + b2_ref[...]
    zp_ref[...] = h.reshape(BV, D, PF).transpose(0, 2, 1).reshape(N, D)
    cb = cb_ref[...]
    c2_ref[...] = jnp.sum(cb * cb, axis=1, keepdims=True)


def _main(zp_ref, cb_ref, c2_ref, mu_ref, sig_ref, Wf_ref, bf_ref, fg_ref,
          fb_ref, Wd_ref, bd_ref, logits_ref, recon_ref):
    z = zp_ref[...]
    cb = cb_ref[...]
    zc = _dotb(z, cb, (((1,), (1,)), ((), ())))
    z2 = jnp.sum(z * z, axis=1, keepdims=True)
    lg = 2.0 * zc - (z2 + c2_ref[...])
    logits_ref[...] = lg
    ms = [jnp.max(lg, axis=1, keepdims=True)]
    for _ in range(TOPK - 1):
        ms.append(jnp.max(jnp.where(lg < ms[-1], lg, -jnp.inf),
                          axis=1, keepdims=True))
    m1, m5 = ms[0], ms[-1]
    wraw = jnp.where(lg >= m5, jnp.exp(lg - m1), 0.0)
    denom = sum(jnp.exp(mj - m1) for mj in ms)
    z_code = _dotb(wraw, cb) / denom
    f = z_code + jax.nn.relu(_dotb(z, Wf_ref[...]) + bf_ref[...])
    m = jnp.mean(f, axis=1, keepdims=True)
    va = jnp.mean((f - m) ** 2, axis=1, keepdims=True)
    fz = (f - m) / jnp.sqrt(va + 1e-5) * fg_ref[...] + fb_ref[...]
    rp = _dotb(fz, Wd_ref[...]) + bd_ref[...]
    recon_ref[...] = rp * sig_ref[...] + mu_ref[...]


def kernel(x, codebook, W_enc, b_enc, ln_g, ln_b, W1, b1, Wmid, bmid, W2, b2,
           Wf, bf, fuse_g, fuse_b, W_dec, b_dec):
    r1 = lambda a: a.reshape(1, -1)
    x2 = jnp.transpose(x, (0, 2, 1)).reshape(BV, L)
    xn, mu, sig = _pallas_call(
        _revin,
        out_shape=[
            jax.ShapeDtypeStruct((BV, L), jnp.float32),
            jax.ShapeDtypeStruct((BV, 1), jnp.float32),
            jax.ShapeDtypeStruct((BV, 1), jnp.float32),
        ],
    )(x2)
    patches = xn.reshape(BV * PH, P)
    zp, c2col = _pallas_call(
        _front,
        out_shape=[
            jax.ShapeDtypeStruct((N, D), jnp.float32),
            jax.ShapeDtypeStruct((K, 1), jnp.float32),
        ],
    )(patches, W_enc, r1(b_enc), r1(ln_g), r1(ln_b), W1, r1(b1), Wmid,
      r1(bmid), W2, r1(b2), codebook)
    c2row = c2col.reshape(1, K)
    mu_rep = jnp.repeat(mu, PF, axis=0)
    sig_rep = jnp.repeat(sig, PF, axis=0)
    logits, recon_pre = _pallas_call(
        _main,
        grid=(NBLK,),
        in_specs=[
            pl.BlockSpec((RB, D), lambda i: (i, 0)),
            pl.BlockSpec((K, D), lambda i: (0, 0)),
            pl.BlockSpec((1, K), lambda i: (0, 0)),
            pl.BlockSpec((RB, 1), lambda i: (i, 0)),
            pl.BlockSpec((RB, 1), lambda i: (i, 0)),
            pl.BlockSpec((D, D), lambda i: (0, 0)),
            pl.BlockSpec((1, D), lambda i: (0, 0)),
            pl.BlockSpec((1, D), lambda i: (0, 0)),
            pl.BlockSpec((1, D), lambda i: (0, 0)),
            pl.BlockSpec((D, P), lambda i: (0, 0)),
            pl.BlockSpec((1, P), lambda i: (0, 0)),
        ],
        out_specs=[
            pl.BlockSpec((RB, K), lambda i: (i, 0)),
            pl.BlockSpec((RB, P), lambda i: (i, 0)),
        ],
        out_shape=[
            jax.ShapeDtypeStruct((N, K), jnp.float32),
            jax.ShapeDtypeStruct((N, P), jnp.float32),
        ],
        compiler_params=pltpu.CompilerParams(
            dimension_semantics=("parallel",)),
    )(zp, codebook, c2row, mu_rep, sig_rep, Wf, r1(bf), r1(fuse_g),
      r1(fuse_b), W_dec, r1(b_dec))
    recon = recon_pre.reshape(B, V, L).transpose(0, 2, 1)
    return recon, logits.reshape(B, V, PF, K)
